# baseline (speedup 1.0000x reference)
import functools

import numpy as np
import jax
import jax.numpy as jnp
from jax.experimental import pallas as pl
from jax.experimental.pallas import tpu as pltpu

_SCALES = (1, 2, 4)
_PREC = jax.lax.Precision.HIGHEST


def _dft_constants(L: int, s: int):
    Ls = L // s
    H = Ls // 2
    t2 = np.arange(H, dtype=np.float64)
    fh = np.arange(1, H // 2 + 1, dtype=np.float64)
    ang_h = 2.0 * np.pi * np.outer(fh, t2) / H
    cth = np.cos(ang_h).astype(np.float32)
    sth = np.sin(ang_h).astype(np.float32)
    twf = np.arange(1, H // 2 + 1, dtype=np.float64) * (2.0 * np.pi / Ls)
    twc = np.broadcast_to(np.cos(twf)[:, None], (H // 2, 128)).astype(np.float32)
    tws = np.broadcast_to(np.sin(twf)[:, None], (H // 2, 128)).astype(np.float32)
    perm = np.concatenate([np.arange(1, H // 2 + 1),
                           np.arange(Ls // 2 - 1, H // 2, -1),
                           [Ls // 2]]).astype(np.float64)
    t = np.arange(Ls, dtype=np.float64)
    ang = 2.0 * np.pi * np.outer(t, perm) / Ls
    w = np.where(perm == Ls // 2, 1.0, 2.0)[None, :] / Ls
    cit = (np.cos(ang) * w).astype(np.float32)
    sit = (-np.sin(ang) * w).astype(np.float32)
    return cth, sth, twc, tws, cit, sit


def _rfft_kernel(q_ref, k_ref, cth_ref, sth_ref, twc_ref, tws_ref,
                 qfr_ref, qfi_ref, kfr_ref, kfi_ref, dc_ref, *, s):
    cth = cth_ref[...]
    sth = sth_ref[...]
    dot = functools.partial(jax.lax.dot, precision=_PREC,
                            preferred_element_type=jnp.float32)

    def fwd(x, xr_ref, xi_ref):
        L, CT = x.shape
        Ls = L // s
        H = Ls // 2
        if s > 1:
            x = jnp.mean(x.reshape(Ls, s, CT), axis=1)
        x2 = x.reshape(H, 2, CT)
        xe = x2[:, 0, :]
        xo = x2[:, 1, :]
        er = dot(cth, xe)
        ei = -dot(sth, xe)
        orr = dot(cth, xo)
        oi = -dot(sth, xo)
        twc = twc_ref[...][:, 0:1]
        tws = tws_ref[...][:, 0:1]
        wor = twc * orr + tws * oi
        woi = twc * oi - tws * orr
        e0 = jnp.sum(xe, axis=0, keepdims=True)
        o0 = jnp.sum(xo, axis=0, keepdims=True)
        m = H // 2 - 1
        xr_ref[0] = jnp.concatenate(
            [er + wor, er[:m] - wor[:m], e0 - o0], axis=0)
        xi_ref[0] = jnp.concatenate(
            [ei + woi, woi[:m] - ei[:m], jnp.zeros_like(e0)], axis=0)
        return e0 + o0

    qdc = fwd(q_ref[0], qfr_ref, qfi_ref)
    kdc = fwd(k_ref[0], kfr_ref, kfi_ref)
    dc_ref[0] = jnp.broadcast_to(qdc * kdc, dc_ref.shape[1:])


def _icorr_kernel(qfr_ref, qfi_ref, kfr_ref, kfi_ref, dc_ref,
                  cit_ref, sit_ref, corr_ref, *, inv_ls):
    qfr = qfr_ref[0]
    qfi = qfi_ref[0]
    kfr = kfr_ref[0]
    kfi = kfi_ref[0]
    pre = qfr * kfr + qfi * kfi
    pim = qfi * kfr - qfr * kfi
    dot = functools.partial(jax.lax.dot, precision=_PREC,
                            preferred_element_type=jnp.float32)
    dc = dc_ref[0][0:1, :] * inv_ls
    corr_ref[0] = dot(cit_ref[...], pre) + dot(sit_ref[...], pim) + dc


def _corr(q3, k3, s):
    B, L, C = q3.shape
    Ls = L // s
    FP = Ls // 2
    H = Ls // 2
    cth, sth, twc, tws, cit, sit = _dft_constants(L, s)

    CT = 256
    NC = C // CT
    NM2 = max(1, Ls // 512)
    MT2 = Ls // NM2

    freq = jax.ShapeDtypeStruct((B, FP, C), jnp.float32)
    qfr, qfi, kfr, kfi, dc = pl.pallas_call(
        functools.partial(_rfft_kernel, s=s),
        grid=(B, NC),
        in_specs=[
            pl.BlockSpec((1, L, CT), lambda b, j: (b, 0, j)),
            pl.BlockSpec((1, L, CT), lambda b, j: (b, 0, j)),
            pl.BlockSpec((H // 2, H), lambda b, j: (0, 0)),
            pl.BlockSpec((H // 2, H), lambda b, j: (0, 0)),
            pl.BlockSpec((H // 2, 128), lambda b, j: (0, 0)),
            pl.BlockSpec((H // 2, 128), lambda b, j: (0, 0)),
        ],
        out_specs=[pl.BlockSpec((1, FP, CT), lambda b, j: (b, 0, j))] * 4
        + [pl.BlockSpec((1, 8, CT), lambda b, j: (b, 0, j))],
        out_shape=[freq] * 4
        + [jax.ShapeDtypeStruct((B, 8, C), jnp.float32)],
    )(q3, k3, cth, sth, twc, tws)
    corr = pl.pallas_call(
        functools.partial(_icorr_kernel, inv_ls=1.0 / Ls),
        grid=(B, NC, NM2),
        in_specs=[pl.BlockSpec((1, FP, CT), lambda b, j, m: (b, 0, j))] * 4
        + [pl.BlockSpec((1, 8, CT), lambda b, j, m: (b, 0, j))]
        + [
            pl.BlockSpec((MT2, FP), lambda b, j, m: (m, 0)),
            pl.BlockSpec((MT2, FP), lambda b, j, m: (m, 0)),
        ],
        out_specs=pl.BlockSpec((1, MT2, CT), lambda b, j, m: (b, m, j)),
        out_shape=jax.ShapeDtypeStruct((B, Ls, C), jnp.float32),
    )(qfr, qfi, kfr, kfi, dc, cit, sit)
    return corr


def _thresh_kernel(corr_ref, aw_ref, okey_scr, *, ksel):
    R = corr_ref[0]
    Ls, C = R.shape
    int_min = jnp.int32(-2147483648)

    def peaks(x):
        idx = jax.lax.broadcasted_iota(jnp.int32, (Ls, C), 0)
        return ((x > jnp.roll(x, 1, axis=0)) & (x > jnp.roll(x, -1, axis=0))
                & (idx >= 1) & (idx <= Ls - 2))

    i = jax.lax.bitcast_convert_type(R, jnp.int32)
    okey_scr[...] = jnp.where(
        peaks(R), jnp.where(i >= 0, i, i ^ jnp.int32(0x7FFFFFFF)), int_min)

    def avg(a, b):
        return (a >> 1) + (b >> 1) + (a & b & 1)

    def body(_, st8):
        lo, hi, cnt_lo, cnt_hi = st8
        m2 = avg(lo, hi)
        m1 = avg(lo, m2)
        m3 = avg(m2, hi)
        o = okey_scr[...]
        c1 = jnp.sum((o >= m1).astype(jnp.int32))
        c2 = jnp.sum((o >= m2).astype(jnp.int32))
        c3 = jnp.sum((o >= m3).astype(jnp.int32))
        ge3 = c3 >= ksel
        ge2 = c2 >= ksel
        ge1 = c1 >= ksel
        lo2 = jnp.where(ge3, m3, jnp.where(ge2, m2, jnp.where(ge1, m1, lo)))
        hi2 = jnp.where(ge3, hi, jnp.where(ge2, m3, jnp.where(ge1, m2, m1)))
        cl2 = jnp.where(ge3, c3, jnp.where(ge2, c2, jnp.where(ge1, c1, cnt_lo)))
        ch2 = jnp.where(ge3, cnt_hi, jnp.where(ge2, c3, jnp.where(ge1, c2, c1)))
        return lo2, hi2, cl2, ch2

    tau, _, cnt_ge, cnt_gt = jax.lax.fori_loop(
        0, 17, body, (int_min, jnp.int32(2147483647),
                      jnp.int32(Ls * C), jnp.int32(0)))
    t_need = ksel - cnt_gt

    def pos():
        return (jax.lax.broadcasted_iota(jnp.int32, (Ls, C), 0) * C
                + jax.lax.broadcasted_iota(jnp.int32, (Ls, C), 1))

    def tie_cut():
        def tbody(_, lohi):
            lo, hi = lohi
            mid = (lo + hi) // 2
            c = jnp.sum(((okey_scr[...] == tau) & (pos() < mid))
                        .astype(jnp.int32))
            ge = c >= t_need
            return jnp.where(ge, lo, mid), jnp.where(ge, mid, hi)

        nbits = max(1, (Ls * C).bit_length())
        _, p0 = jax.lax.fori_loop(0, nbits, tbody,
                                  (jnp.int32(0), jnp.int32(Ls * C)))
        return p0

    p0 = jax.lax.cond(cnt_ge == ksel, lambda: jnp.int32(Ls * C), tie_cut)
    tie_sel = (okey_scr[...] == tau) & (pos() < p0) & peaks(R)
    aw_ref[0] = jnp.where((okey_scr[...] > tau) | tie_sel, R, 0.0)


def _weight_kernel(aw_ref, v_ref, out_ref, *, s):
    aw = aw_ref[0]
    Ls, CT = aw.shape
    v = v_ref[0]
    if s > 1:
        v = jnp.mean(v.reshape(Ls, s, CT), axis=1)
    mx = jnp.max(aw, axis=0, keepdims=True)
    e = jnp.exp(aw - mx)
    den = jnp.sum(e, axis=0, keepdims=True)
    out_ref[0] = (e / den) * v


def _select_agg(corr, v3, s, ksel):
    B, Ls, C = corr.shape
    L = v3.shape[1]
    aw = pl.pallas_call(
        functools.partial(_thresh_kernel, ksel=ksel),
        grid=(B,),
        in_specs=[pl.BlockSpec((1, Ls, C), lambda b: (b, 0, 0))],
        out_specs=pl.BlockSpec((1, Ls, C), lambda b: (b, 0, 0)),
        out_shape=jax.ShapeDtypeStruct((B, Ls, C), jnp.float32),
        scratch_shapes=[pltpu.VMEM((Ls, C), jnp.int32)],
    )(corr)
    CT = 256
    return pl.pallas_call(
        functools.partial(_weight_kernel, s=s),
        grid=(B, C // CT),
        in_specs=[pl.BlockSpec((1, Ls, CT), lambda b, j: (b, 0, j)),
                  pl.BlockSpec((1, L, CT), lambda b, j: (b, 0, j))],
        out_specs=pl.BlockSpec((1, Ls, CT), lambda b, j: (b, 0, j)),
        out_shape=jax.ShapeDtypeStruct((B, Ls, C), jnp.float32),
    )(aw, v3)


def _up2(y):
    yp = jnp.concatenate([y[:1], y[:-1]], axis=0)
    yn = jnp.concatenate([y[1:], y[-1:]], axis=0)
    even = 0.25 * yp + 0.75 * y
    odd = 0.75 * y + 0.25 * yn
    Ls, CT = y.shape
    return jnp.stack([even, odd], axis=1).reshape(2 * Ls, CT)


def _up4(y):
    yp = jnp.concatenate([y[:1], y[:-1]], axis=0)
    yn = jnp.concatenate([y[1:], y[-1:]], axis=0)
    p0 = 0.375 * yp + 0.625 * y
    p1 = 0.125 * yp + 0.875 * y
    p2 = 0.875 * y + 0.125 * yn
    p3 = 0.625 * y + 0.375 * yn
    Ls, CT = y.shape
    return jnp.stack([p0, p1, p2, p3], axis=1).reshape(4 * Ls, CT)


def _mix_kernel(y1_ref, y2_ref, y4_ref, sw_ref, out_ref):
    sw = sw_ref[...]
    out_ref[0] = (sw[0, 0, 0] * y1_ref[0]
                  + sw[1, 0, 0] * _up2(y2_ref[0])
                  + sw[2, 0, 0] * _up4(y4_ref[0]))


def _mix(y1, y2, y4, scale_weights):
    B, L, C = y1.shape
    swb = jnp.broadcast_to(scale_weights.reshape(3, 1, 1), (3, 8, 128))
    CT = 256
    return pl.pallas_call(
        _mix_kernel,
        grid=(B, C // CT),
        in_specs=[
            pl.BlockSpec((1, L, CT), lambda b, j: (b, 0, j)),
            pl.BlockSpec((1, L // 2, CT), lambda b, j: (b, 0, j)),
            pl.BlockSpec((1, L // 4, CT), lambda b, j: (b, 0, j)),
            pl.BlockSpec((3, 8, 128), lambda b, j: (0, 0, 0)),
        ],
        out_specs=pl.BlockSpec((1, L, CT), lambda b, j: (b, 0, j)),
        out_shape=jax.ShapeDtypeStruct((B, L, C), jnp.float32),
    )(y1, y2, y4, swb)


def kernel(queries, keys, values, attn_mask, scale_weights):
    B, L, H, E = queries.shape
    C = H * E
    q3 = queries.reshape(B, L, C)
    k3 = keys.reshape(B, L, C)
    v3 = values.reshape(B, L, C)
    ys = []
    for s in _SCALES:
        corr = _corr(q3, k3, s)
        ys.append(_select_agg(corr, v3, s, ksel=L // s))
    total = _mix(ys[0], ys[1], ys[2], scale_weights)
    return total.reshape(B, L, H, E)


# speedup vs baseline: 1.0757x; 1.0757x over previous
import functools

import numpy as np
import jax
import jax.numpy as jnp
from jax.experimental import pallas as pl
from jax.experimental.pallas import tpu as pltpu

_SCALES = (1, 2, 4)
_PREC = jax.lax.Precision.HIGHEST


def _dft_constants(L: int, s: int):
    Ls = L // s
    H = Ls // 2
    t2 = np.arange(H, dtype=np.float64)
    fh = np.arange(1, H // 2 + 1, dtype=np.float64)
    ang_h = 2.0 * np.pi * np.outer(fh, t2) / H
    cth = np.cos(ang_h).astype(np.float32)
    sth = np.sin(ang_h).astype(np.float32)
    twf = np.arange(1, H // 2 + 1, dtype=np.float64) * (2.0 * np.pi / Ls)
    twc = np.broadcast_to(np.cos(twf)[:, None], (H // 2, 128)).astype(np.float32)
    tws = np.broadcast_to(np.sin(twf)[:, None], (H // 2, 128)).astype(np.float32)
    M = Ls // 2
    m = np.arange(M, dtype=np.float64)
    g = np.arange(1, M // 2 + 1, dtype=np.float64)
    ang_i = 2.0 * np.pi * np.outer(m, g) / M
    wi = np.where(g == M // 2, 1.0, 2.0)[None, :] / Ls
    cie = (np.cos(ang_i) * wi).astype(np.float32)
    sie = (-np.sin(ang_i) * wi).astype(np.float32)
    return cth, sth, twc, tws, cie, sie


def _rfft_kernel(q_ref, k_ref, cth_ref, sth_ref, twc_ref, tws_ref,
                 qfr_ref, qfi_ref, kfr_ref, kfi_ref, dc_ref, *, s):
    cth = cth_ref[...]
    sth = sth_ref[...]
    dot = functools.partial(jax.lax.dot, precision=_PREC,
                            preferred_element_type=jnp.float32)

    def fwd(x, xr_ref, xi_ref):
        L, CT = x.shape
        Ls = L // s
        H = Ls // 2
        if s > 1:
            x = jnp.mean(x.reshape(Ls, s, CT), axis=1)
        x2 = x.reshape(H, 2, CT)
        xe = x2[:, 0, :]
        xo = x2[:, 1, :]
        er = dot(cth, xe)
        ei = -dot(sth, xe)
        orr = dot(cth, xo)
        oi = -dot(sth, xo)
        twc = twc_ref[...][:, 0:1]
        tws = tws_ref[...][:, 0:1]
        wor = twc * orr + tws * oi
        woi = twc * oi - tws * orr
        e0 = jnp.sum(xe, axis=0, keepdims=True)
        o0 = jnp.sum(xo, axis=0, keepdims=True)
        m = H // 2 - 1
        xr_ref[0] = jnp.concatenate(
            [er + wor, er[:m] - wor[:m], e0 - o0], axis=0)
        xi_ref[0] = jnp.concatenate(
            [ei + woi, woi[:m] - ei[:m], jnp.zeros_like(e0)], axis=0)
        return e0 + o0

    qdc = fwd(q_ref[0], qfr_ref, qfi_ref)
    kdc = fwd(k_ref[0], kfr_ref, kfi_ref)
    dc_ref[0] = jnp.broadcast_to(qdc * kdc, dc_ref.shape[1:])


def _icorr_kernel(qfr_ref, qfi_ref, kfr_ref, kfi_ref, dc_ref,
                  cie_ref, sie_ref, twc_ref, tws_ref, corr_ref, *, inv_ls):
    qfr = qfr_ref[0]
    qfi = qfi_ref[0]
    kfr = kfr_ref[0]
    kfi = kfi_ref[0]
    pre = qfr * kfr + qfi * kfi
    pim = qfi * kfr - qfr * kfi
    FP, CT = pre.shape
    FPh = FP // 2
    dot = functools.partial(jax.lax.dot, precision=_PREC,
                            preferred_element_type=jnp.float32)
    a_r = pre[0:FPh - 1]
    b_r = pre[FPh:FP - 1]
    a_i = pim[0:FPh - 1]
    b_i = pim[FPh:FP - 1]
    ny_r = pre[FPh - 1:FPh]
    ny_i = pim[FPh - 1:FPh]
    p_last = pre[FP - 1:FP]
    dc = dc_ref[0][0:1, :]
    z = jnp.zeros_like(ny_r)
    gr = jnp.concatenate([a_r + b_r, 2.0 * ny_r], axis=0)
    gi = jnp.concatenate([a_i - b_i, z], axis=0)
    dr = jnp.concatenate([a_r - b_r, z], axis=0)
    di = jnp.concatenate([a_i + b_i, 2.0 * ny_i], axis=0)
    twc = twc_ref[...][:, 0:1]
    tws = tws_ref[...][:, 0:1]
    gor = twc * dr - tws * di
    goi = twc * di + tws * dr
    even = dot(cie_ref[...], gr) + dot(sie_ref[...], gi) \
        + (dc + p_last) * inv_ls
    odd = dot(cie_ref[...], gor) + dot(sie_ref[...], goi) \
        + (dc - p_last) * inv_ls
    M = even.shape[0]
    corr_ref[0] = jnp.stack([even, odd], axis=1).reshape(2 * M, CT)


def _corr(q3, k3, s):
    B, L, C = q3.shape
    Ls = L // s
    FP = Ls // 2
    H = Ls // 2
    cth, sth, twc, tws, cie, sie = _dft_constants(L, s)

    CT = 256
    NC = C // CT
    M = Ls // 2

    freq = jax.ShapeDtypeStruct((B, FP, C), jnp.float32)
    qfr, qfi, kfr, kfi, dc = pl.pallas_call(
        functools.partial(_rfft_kernel, s=s),
        grid=(B, NC),
        in_specs=[
            pl.BlockSpec((1, L, CT), lambda b, j: (b, 0, j)),
            pl.BlockSpec((1, L, CT), lambda b, j: (b, 0, j)),
            pl.BlockSpec((H // 2, H), lambda b, j: (0, 0)),
            pl.BlockSpec((H // 2, H), lambda b, j: (0, 0)),
            pl.BlockSpec((H // 2, 128), lambda b, j: (0, 0)),
            pl.BlockSpec((H // 2, 128), lambda b, j: (0, 0)),
        ],
        out_specs=[pl.BlockSpec((1, FP, CT), lambda b, j: (b, 0, j))] * 4
        + [pl.BlockSpec((1, 8, CT), lambda b, j: (b, 0, j))],
        out_shape=[freq] * 4
        + [jax.ShapeDtypeStruct((B, 8, C), jnp.float32)],
    )(q3, k3, cth, sth, twc, tws)
    corr = pl.pallas_call(
        functools.partial(_icorr_kernel, inv_ls=1.0 / Ls),
        grid=(B, NC),
        in_specs=[pl.BlockSpec((1, FP, CT), lambda b, j: (b, 0, j))] * 4
        + [pl.BlockSpec((1, 8, CT), lambda b, j: (b, 0, j))]
        + [
            pl.BlockSpec((M, M // 2), lambda b, j: (0, 0)),
            pl.BlockSpec((M, M // 2), lambda b, j: (0, 0)),
            pl.BlockSpec((M // 2, 128), lambda b, j: (0, 0)),
            pl.BlockSpec((M // 2, 128), lambda b, j: (0, 0)),
        ],
        out_specs=pl.BlockSpec((1, Ls, CT), lambda b, j: (b, 0, j)),
        out_shape=jax.ShapeDtypeStruct((B, Ls, C), jnp.float32),
    )(qfr, qfi, kfr, kfi, dc, cie, sie, twc, tws)
    return corr


def _thresh_kernel(corr_ref, aw_ref, okey_scr, *, ksel):
    R = corr_ref[0]
    Ls, C = R.shape
    int_min = jnp.int32(-2147483648)

    def peaks(x):
        idx = jax.lax.broadcasted_iota(jnp.int32, (Ls, C), 0)
        return ((x > jnp.roll(x, 1, axis=0)) & (x > jnp.roll(x, -1, axis=0))
                & (idx >= 1) & (idx <= Ls - 2))

    i = jax.lax.bitcast_convert_type(R, jnp.int32)
    okey_scr[...] = jnp.where(
        peaks(R), jnp.where(i >= 0, i, i ^ jnp.int32(0x7FFFFFFF)), int_min)

    def avg(a, b):
        return (a >> 1) + (b >> 1) + (a & b & 1)

    def body(_, st8):
        lo, hi, cnt_lo, cnt_hi = st8
        m2 = avg(lo, hi)
        m1 = avg(lo, m2)
        m3 = avg(m2, hi)
        o = okey_scr[...]
        c1 = jnp.sum((o >= m1).astype(jnp.int32))
        c2 = jnp.sum((o >= m2).astype(jnp.int32))
        c3 = jnp.sum((o >= m3).astype(jnp.int32))
        ge3 = c3 >= ksel
        ge2 = c2 >= ksel
        ge1 = c1 >= ksel
        lo2 = jnp.where(ge3, m3, jnp.where(ge2, m2, jnp.where(ge1, m1, lo)))
        hi2 = jnp.where(ge3, hi, jnp.where(ge2, m3, jnp.where(ge1, m2, m1)))
        cl2 = jnp.where(ge3, c3, jnp.where(ge2, c2, jnp.where(ge1, c1, cnt_lo)))
        ch2 = jnp.where(ge3, cnt_hi, jnp.where(ge2, c3, jnp.where(ge1, c2, c1)))
        return lo2, hi2, cl2, ch2

    tau, _, cnt_ge, cnt_gt = jax.lax.fori_loop(
        0, 17, body, (int_min, jnp.int32(2147483647),
                      jnp.int32(Ls * C), jnp.int32(0)))
    t_need = ksel - cnt_gt

    def pos():
        return (jax.lax.broadcasted_iota(jnp.int32, (Ls, C), 0) * C
                + jax.lax.broadcasted_iota(jnp.int32, (Ls, C), 1))

    def tie_cut():
        def tbody(_, lohi):
            lo, hi = lohi
            mid = (lo + hi) // 2
            c = jnp.sum(((okey_scr[...] == tau) & (pos() < mid))
                        .astype(jnp.int32))
            ge = c >= t_need
            return jnp.where(ge, lo, mid), jnp.where(ge, mid, hi)

        nbits = max(1, (Ls * C).bit_length())
        _, p0 = jax.lax.fori_loop(0, nbits, tbody,
                                  (jnp.int32(0), jnp.int32(Ls * C)))
        return p0

    p0 = jax.lax.cond(cnt_ge == ksel, lambda: jnp.int32(Ls * C), tie_cut)
    tie_sel = (okey_scr[...] == tau) & (pos() < p0) & peaks(R)
    aw_ref[0] = jnp.where((okey_scr[...] > tau) | tie_sel, R, 0.0)


def _weight_kernel(aw_ref, v_ref, out_ref, *, s):
    aw = aw_ref[0]
    Ls, CT = aw.shape
    v = v_ref[0]
    if s > 1:
        v = jnp.mean(v.reshape(Ls, s, CT), axis=1)
    mx = jnp.max(aw, axis=0, keepdims=True)
    e = jnp.exp(aw - mx)
    den = jnp.sum(e, axis=0, keepdims=True)
    out_ref[0] = (e / den) * v


def _select_agg(corr, v3, s, ksel):
    B, Ls, C = corr.shape
    L = v3.shape[1]
    aw = pl.pallas_call(
        functools.partial(_thresh_kernel, ksel=ksel),
        grid=(B,),
        in_specs=[pl.BlockSpec((1, Ls, C), lambda b: (b, 0, 0))],
        out_specs=pl.BlockSpec((1, Ls, C), lambda b: (b, 0, 0)),
        out_shape=jax.ShapeDtypeStruct((B, Ls, C), jnp.float32),
        scratch_shapes=[pltpu.VMEM((Ls, C), jnp.int32)],
    )(corr)
    CT = 256
    return pl.pallas_call(
        functools.partial(_weight_kernel, s=s),
        grid=(B, C // CT),
        in_specs=[pl.BlockSpec((1, Ls, CT), lambda b, j: (b, 0, j)),
                  pl.BlockSpec((1, L, CT), lambda b, j: (b, 0, j))],
        out_specs=pl.BlockSpec((1, Ls, CT), lambda b, j: (b, 0, j)),
        out_shape=jax.ShapeDtypeStruct((B, Ls, C), jnp.float32),
    )(aw, v3)


def _up2(y):
    yp = jnp.concatenate([y[:1], y[:-1]], axis=0)
    yn = jnp.concatenate([y[1:], y[-1:]], axis=0)
    even = 0.25 * yp + 0.75 * y
    odd = 0.75 * y + 0.25 * yn
    Ls, CT = y.shape
    return jnp.stack([even, odd], axis=1).reshape(2 * Ls, CT)


def _up4(y):
    yp = jnp.concatenate([y[:1], y[:-1]], axis=0)
    yn = jnp.concatenate([y[1:], y[-1:]], axis=0)
    p0 = 0.375 * yp + 0.625 * y
    p1 = 0.125 * yp + 0.875 * y
    p2 = 0.875 * y + 0.125 * yn
    p3 = 0.625 * y + 0.375 * yn
    Ls, CT = y.shape
    return jnp.stack([p0, p1, p2, p3], axis=1).reshape(4 * Ls, CT)


def _mix_kernel(y1_ref, y2_ref, y4_ref, sw_ref, out_ref):
    sw = sw_ref[...]
    out_ref[0] = (sw[0, 0, 0] * y1_ref[0]
                  + sw[1, 0, 0] * _up2(y2_ref[0])
                  + sw[2, 0, 0] * _up4(y4_ref[0]))


def _mix(y1, y2, y4, scale_weights):
    B, L, C = y1.shape
    swb = jnp.broadcast_to(scale_weights.reshape(3, 1, 1), (3, 8, 128))
    CT = 256
    return pl.pallas_call(
        _mix_kernel,
        grid=(B, C // CT),
        in_specs=[
            pl.BlockSpec((1, L, CT), lambda b, j: (b, 0, j)),
            pl.BlockSpec((1, L // 2, CT), lambda b, j: (b, 0, j)),
            pl.BlockSpec((1, L // 4, CT), lambda b, j: (b, 0, j)),
            pl.BlockSpec((3, 8, 128), lambda b, j: (0, 0, 0)),
        ],
        out_specs=pl.BlockSpec((1, L, CT), lambda b, j: (b, 0, j)),
        out_shape=jax.ShapeDtypeStruct((B, L, C), jnp.float32),
    )(y1, y2, y4, swb)


def kernel(queries, keys, values, attn_mask, scale_weights):
    B, L, H, E = queries.shape
    C = H * E
    q3 = queries.reshape(B, L, C)
    k3 = keys.reshape(B, L, C)
    v3 = values.reshape(B, L, C)
    ys = []
    for s in _SCALES:
        corr = _corr(q3, k3, s)
        ys.append(_select_agg(corr, v3, s, ksel=L // s))
    total = _mix(ys[0], ys[1], ys[2], scale_weights)
    return total.reshape(B, L, H, E)


# speedup vs baseline: 1.0801x; 1.0041x over previous
import functools

import numpy as np
import jax
import jax.numpy as jnp
from jax.experimental import pallas as pl
from jax.experimental.pallas import tpu as pltpu

_SCALES = (1, 2, 4)
_PREC = jax.lax.Precision.HIGHEST


def _dft_constants(L: int, s: int):
    Ls = L // s
    H = Ls // 2
    t2 = np.arange(H, dtype=np.float64)
    fh = np.arange(1, H // 2 + 1, dtype=np.float64)
    ang_h = 2.0 * np.pi * np.outer(fh, t2) / H
    cth = np.cos(ang_h).astype(np.float32)
    sth = np.sin(ang_h).astype(np.float32)
    twf = np.arange(1, H // 2 + 1, dtype=np.float64) * (2.0 * np.pi / Ls)
    twc = np.broadcast_to(np.cos(twf)[:, None], (H // 2, 128)).astype(np.float32)
    tws = np.broadcast_to(np.sin(twf)[:, None], (H // 2, 128)).astype(np.float32)
    M = Ls // 2
    m = np.arange(M, dtype=np.float64)
    g = np.arange(1, M // 2 + 1, dtype=np.float64)
    ang_i = 2.0 * np.pi * np.outer(m, g) / M
    wi = np.where(g == M // 2, 1.0, 2.0)[None, :] / Ls
    cie = (np.cos(ang_i) * wi).astype(np.float32)
    sie = (-np.sin(ang_i) * wi).astype(np.float32)
    return cth, sth, twc, tws, cie, sie


def _rfft_kernel(q_ref, k_ref, cth_ref, sth_ref, twc_ref, tws_ref,
                 qfr_ref, qfi_ref, kfr_ref, kfi_ref, dc_ref, *, s):
    cth = cth_ref[...]
    sth = sth_ref[...]
    dot = functools.partial(jax.lax.dot, precision=_PREC,
                            preferred_element_type=jnp.float32)

    def fwd(x, xr_ref, xi_ref):
        L, CT = x.shape
        Ls = L // s
        H = Ls // 2
        if s > 1:
            x = jnp.mean(x.reshape(Ls, s, CT), axis=1)
        x2 = x.reshape(H, 2, CT)
        xe = x2[:, 0, :]
        xo = x2[:, 1, :]
        er = dot(cth, xe)
        ei = -dot(sth, xe)
        orr = dot(cth, xo)
        oi = -dot(sth, xo)
        twc = twc_ref[...][:, 0:1]
        tws = tws_ref[...][:, 0:1]
        wor = twc * orr + tws * oi
        woi = twc * oi - tws * orr
        e0 = jnp.sum(xe, axis=0, keepdims=True)
        o0 = jnp.sum(xo, axis=0, keepdims=True)
        m = H // 2 - 1
        xr_ref[0] = jnp.concatenate(
            [er + wor, er[:m] - wor[:m], e0 - o0], axis=0)
        xi_ref[0] = jnp.concatenate(
            [ei + woi, woi[:m] - ei[:m], jnp.zeros_like(e0)], axis=0)
        return e0 + o0

    qdc = fwd(q_ref[0], qfr_ref, qfi_ref)
    kdc = fwd(k_ref[0], kfr_ref, kfi_ref)
    dc_ref[0] = jnp.broadcast_to(qdc * kdc, dc_ref.shape[1:])


def _icorr_kernel(qfr_ref, qfi_ref, kfr_ref, kfi_ref, dc_ref,
                  cie_ref, sie_ref, twc_ref, tws_ref, corr_ref, *, inv_ls):
    qfr = qfr_ref[0]
    qfi = qfi_ref[0]
    kfr = kfr_ref[0]
    kfi = kfi_ref[0]
    pre = qfr * kfr + qfi * kfi
    pim = qfi * kfr - qfr * kfi
    FP, CT = pre.shape
    FPh = FP // 2
    dot = functools.partial(jax.lax.dot, precision=_PREC,
                            preferred_element_type=jnp.float32)
    a_r = pre[0:FPh - 1]
    b_r = pre[FPh:FP - 1]
    a_i = pim[0:FPh - 1]
    b_i = pim[FPh:FP - 1]
    ny_r = pre[FPh - 1:FPh]
    ny_i = pim[FPh - 1:FPh]
    p_last = pre[FP - 1:FP]
    dc = dc_ref[0][0:1, :]
    z = jnp.zeros_like(ny_r)
    gr = jnp.concatenate([a_r + b_r, 2.0 * ny_r], axis=0)
    gi = jnp.concatenate([a_i - b_i, z], axis=0)
    dr = jnp.concatenate([a_r - b_r, z], axis=0)
    di = jnp.concatenate([a_i + b_i, 2.0 * ny_i], axis=0)
    twc = twc_ref[...][:, 0:1]
    tws = tws_ref[...][:, 0:1]
    gor = twc * dr - tws * di
    goi = twc * di + tws * dr
    even = dot(cie_ref[...], gr) + dot(sie_ref[...], gi) \
        + (dc + p_last) * inv_ls
    odd = dot(cie_ref[...], gor) + dot(sie_ref[...], goi) \
        + (dc - p_last) * inv_ls
    M = even.shape[0]
    corr_ref[0] = jnp.stack([even, odd], axis=1).reshape(2 * M, CT)


def _corr(q3, k3, s):
    B, L, C = q3.shape
    Ls = L // s
    FP = Ls // 2
    H = Ls // 2
    cth, sth, twc, tws, cie, sie = _dft_constants(L, s)

    CT = 256
    NC = C // CT
    M = Ls // 2

    freq = jax.ShapeDtypeStruct((B, FP, C), jnp.float32)
    qfr, qfi, kfr, kfi, dc = pl.pallas_call(
        functools.partial(_rfft_kernel, s=s),
        grid=(B, NC),
        in_specs=[
            pl.BlockSpec((1, L, CT), lambda b, j: (b, 0, j)),
            pl.BlockSpec((1, L, CT), lambda b, j: (b, 0, j)),
            pl.BlockSpec((H // 2, H), lambda b, j: (0, 0)),
            pl.BlockSpec((H // 2, H), lambda b, j: (0, 0)),
            pl.BlockSpec((H // 2, 128), lambda b, j: (0, 0)),
            pl.BlockSpec((H // 2, 128), lambda b, j: (0, 0)),
        ],
        out_specs=[pl.BlockSpec((1, FP, CT), lambda b, j: (b, 0, j))] * 4
        + [pl.BlockSpec((1, 8, CT), lambda b, j: (b, 0, j))],
        out_shape=[freq] * 4
        + [jax.ShapeDtypeStruct((B, 8, C), jnp.float32)],
    )(q3, k3, cth, sth, twc, tws)
    corr = pl.pallas_call(
        functools.partial(_icorr_kernel, inv_ls=1.0 / Ls),
        grid=(B, NC),
        in_specs=[pl.BlockSpec((1, FP, CT), lambda b, j: (b, 0, j))] * 4
        + [pl.BlockSpec((1, 8, CT), lambda b, j: (b, 0, j))]
        + [
            pl.BlockSpec((M, M // 2), lambda b, j: (0, 0)),
            pl.BlockSpec((M, M // 2), lambda b, j: (0, 0)),
            pl.BlockSpec((M // 2, 128), lambda b, j: (0, 0)),
            pl.BlockSpec((M // 2, 128), lambda b, j: (0, 0)),
        ],
        out_specs=pl.BlockSpec((1, Ls, CT), lambda b, j: (b, 0, j)),
        out_shape=jax.ShapeDtypeStruct((B, Ls, C), jnp.float32),
    )(qfr, qfi, kfr, kfi, dc, cie, sie, twc, tws)
    return corr


def _thresh_kernel(corr_ref, aw_ref, okey_scr, *, ksel):
    R = corr_ref[0]
    Ls, C = R.shape
    int_min = jnp.int32(-2147483648)

    def peaks(x):
        idx = jax.lax.broadcasted_iota(jnp.int32, (Ls, C), 0)
        return ((x > jnp.roll(x, 1, axis=0)) & (x > jnp.roll(x, -1, axis=0))
                & (idx >= 1) & (idx <= Ls - 2))

    i = jax.lax.bitcast_convert_type(R, jnp.int32)
    okey_scr[...] = jnp.where(
        peaks(R), jnp.where(i >= 0, i, i ^ jnp.int32(0x7FFFFFFF)), int_min)

    def avg(a, b):
        return (a >> 1) + (b >> 1) + (a & b & 1)

    def body(_, st8):
        lo, hi, cnt_lo, cnt_hi = st8
        m2 = avg(lo, hi)
        m1 = avg(lo, m2)
        m3 = avg(m2, hi)
        o = okey_scr[...]
        c1 = jnp.sum((o >= m1).astype(jnp.int32))
        c2 = jnp.sum((o >= m2).astype(jnp.int32))
        c3 = jnp.sum((o >= m3).astype(jnp.int32))
        ge3 = c3 >= ksel
        ge2 = c2 >= ksel
        ge1 = c1 >= ksel
        lo2 = jnp.where(ge3, m3, jnp.where(ge2, m2, jnp.where(ge1, m1, lo)))
        hi2 = jnp.where(ge3, hi, jnp.where(ge2, m3, jnp.where(ge1, m2, m1)))
        cl2 = jnp.where(ge3, c3, jnp.where(ge2, c2, jnp.where(ge1, c1, cnt_lo)))
        ch2 = jnp.where(ge3, cnt_hi, jnp.where(ge2, c3, jnp.where(ge1, c2, c1)))
        return lo2, hi2, cl2, ch2

    tau, _, cnt_ge, cnt_gt = jax.lax.fori_loop(
        0, 17, body, (int_min, jnp.int32(2147483647),
                      jnp.int32(Ls * C), jnp.int32(0)))
    t_need = ksel - cnt_gt

    def pos():
        return (jax.lax.broadcasted_iota(jnp.int32, (Ls, C), 0) * C
                + jax.lax.broadcasted_iota(jnp.int32, (Ls, C), 1))

    def tie_cut():
        def tbody(_, lohi):
            lo, hi = lohi
            mid = (lo + hi) // 2
            c = jnp.sum(((okey_scr[...] == tau) & (pos() < mid))
                        .astype(jnp.int32))
            ge = c >= t_need
            return jnp.where(ge, lo, mid), jnp.where(ge, mid, hi)

        nbits = max(1, (Ls * C).bit_length())
        _, p0 = jax.lax.fori_loop(0, nbits, tbody,
                                  (jnp.int32(0), jnp.int32(Ls * C)))
        return p0

    p0 = jax.lax.cond(cnt_ge == ksel, lambda: jnp.int32(Ls * C), tie_cut)
    tie_sel = (okey_scr[...] == tau) & (pos() < p0) & peaks(R)
    aw_ref[0] = jnp.where((okey_scr[...] > tau) | tie_sel, R, 0.0)


def _thresh(corr, ksel):
    B, Ls, C = corr.shape
    return pl.pallas_call(
        functools.partial(_thresh_kernel, ksel=ksel),
        grid=(B,),
        in_specs=[pl.BlockSpec((1, Ls, C), lambda b: (b, 0, 0))],
        out_specs=pl.BlockSpec((1, Ls, C), lambda b: (b, 0, 0)),
        out_shape=jax.ShapeDtypeStruct((B, Ls, C), jnp.float32),
        scratch_shapes=[pltpu.VMEM((Ls, C), jnp.int32)],
    )(corr)


def _up2(y):
    yp = jnp.concatenate([y[:1], y[:-1]], axis=0)
    yn = jnp.concatenate([y[1:], y[-1:]], axis=0)
    even = 0.25 * yp + 0.75 * y
    odd = 0.75 * y + 0.25 * yn
    Ls, CT = y.shape
    return jnp.stack([even, odd], axis=1).reshape(2 * Ls, CT)


def _up4(y):
    yp = jnp.concatenate([y[:1], y[:-1]], axis=0)
    yn = jnp.concatenate([y[1:], y[-1:]], axis=0)
    p0 = 0.375 * yp + 0.625 * y
    p1 = 0.125 * yp + 0.875 * y
    p2 = 0.875 * y + 0.125 * yn
    p3 = 0.625 * y + 0.375 * yn
    Ls, CT = y.shape
    return jnp.stack([p0, p1, p2, p3], axis=1).reshape(4 * Ls, CT)


def _wmix_kernel(aw1_ref, aw2_ref, aw4_ref, v_ref, sw_ref, out_ref):
    v = v_ref[0]
    L, CT = v.shape

    def softw(aw, s):
        vs = v if s == 1 else jnp.mean(v.reshape(L // s, s, CT), axis=1)
        mx = jnp.max(aw, axis=0, keepdims=True)
        e = jnp.exp(aw - mx)
        den = jnp.sum(e, axis=0, keepdims=True)
        return (e / den) * vs

    y1 = softw(aw1_ref[0], 1)
    y2 = softw(aw2_ref[0], 2)
    y4 = softw(aw4_ref[0], 4)
    sw = sw_ref[...]
    out_ref[0] = (sw[0, 0, 0] * y1
                  + sw[1, 0, 0] * _up2(y2)
                  + sw[2, 0, 0] * _up4(y4))


def _wmix(aw1, aw2, aw4, v3, scale_weights):
    B, L, C = v3.shape
    swb = jnp.broadcast_to(scale_weights.reshape(3, 1, 1), (3, 8, 128))
    CT = 256
    return pl.pallas_call(
        _wmix_kernel,
        grid=(B, C // CT),
        in_specs=[
            pl.BlockSpec((1, L, CT), lambda b, j: (b, 0, j)),
            pl.BlockSpec((1, L // 2, CT), lambda b, j: (b, 0, j)),
            pl.BlockSpec((1, L // 4, CT), lambda b, j: (b, 0, j)),
            pl.BlockSpec((1, L, CT), lambda b, j: (b, 0, j)),
            pl.BlockSpec((3, 8, 128), lambda b, j: (0, 0, 0)),
        ],
        out_specs=pl.BlockSpec((1, L, CT), lambda b, j: (b, 0, j)),
        out_shape=jax.ShapeDtypeStruct((B, L, C), jnp.float32),
    )(aw1, aw2, aw4, v3, swb)


def kernel(queries, keys, values, attn_mask, scale_weights):
    B, L, H, E = queries.shape
    C = H * E
    q3 = queries.reshape(B, L, C)
    k3 = keys.reshape(B, L, C)
    v3 = values.reshape(B, L, C)
    aws = []
    for s in _SCALES:
        corr = _corr(q3, k3, s)
        aws.append(_thresh(corr, ksel=L // s))
    total = _wmix(aws[0], aws[1], aws[2], v3, scale_weights)
    return total.reshape(B, L, H, E)
